# baseline (speedup 1.0000x reference)
import jax
import jax.numpy as jnp
from jax import lax
from jax.experimental import pallas as pl
from jax.experimental.pallas import tpu as pltpu
from jax.experimental.pallas import tpu_sc as plsc

N = 10000
E = 320000
F = 128
NPAD = 10240
NC = 2
NS = 16
NW = NC * NS
E_W = E // NW
CH = 64
EPAD = 240
NCHUNK = (E_W + EPAD) // CH
SB = 8
NQ = NCHUNK // SB
NBUF = 4
ROWS_W = NPAD // NS
DEG_W = NPAD // NS

_MESH = dict(core_axis_name="c", subcore_axis_name="s", num_cores=NC,
             num_subcores=NS)
_PARAMS = dict(
    mesh=plsc.VectorSubcoreMesh(**_MESH),
    compiler_params=pltpu.CompilerParams(needs_layout_passes=False),
)


NCH = 2000


def _rsqrt_nr(x):
    y = plsc.bitcast(jnp.int32(0x5F3759DF) - (plsc.bitcast(x, jnp.int32) >> 1),
                     jnp.float32)
    for _ in range(3):
        y = y * (1.5 - 0.5 * x * y * y)
    return y


def _norm_body(src3_hbm, ew3_hbm, src_hbm, dst_hbm, w_hbm, z1_hbm, out_hbm,
               idx_b, wch_b, dis_v, s_b, d_b, w_b, n_b, degsh, ssem):
    cid = lax.axis_index("c")
    sid = lax.axis_index("s")
    wid = cid * NS + sid

    pltpu.sync_copy(z1_hbm.at[pl.ds(sid * DEG_W, DEG_W)],
                    degsh.at[pl.ds(sid * DEG_W, DEG_W)])
    plsc.subcore_barrier()
    for w in range(2):
        pltpu.sync_copy(src3_hbm.at[2 * sid + w], idx_b)
        pltpu.sync_copy(ew3_hbm.at[2 * sid + w], wch_b)

        def fire(k, carry):
            pltpu.async_copy(wch_b.at[k], degsh.at[idx_b.at[k]], ssem,
                             add=True)
            return carry

        lax.fori_loop(0, NCHUNK, fire, 0)

        def drain(k, carry):
            pltpu.make_async_copy(wch_b.at[0], degsh.at[idx_b.at[0]],
                                  ssem).wait()
            return carry

        lax.fori_loop(0, NCHUNK, drain, 0)
    plsc.subcore_barrier()

    pltpu.sync_copy(degsh.at[pl.ds(sid * DEG_W, DEG_W)], n_b.at[pl.ds(0, DEG_W)])

    def dgrp(g, carry):
        sl = pl.ds(g * 16, 16)
        d = n_b[sl]
        n_b[sl] = jnp.where(d > 0, _rsqrt_nr(jnp.where(d > 0, d, 1.0)), 0.0)
        return carry

    lax.fori_loop(0, DEG_W // 16, dgrp, 0)
    pltpu.sync_copy(n_b.at[pl.ds(0, DEG_W)], degsh.at[pl.ds(sid * DEG_W, DEG_W)])
    plsc.subcore_barrier()
    pltpu.sync_copy(degsh, dis_v)

    def chunk(k, carry):
        base = wid * E_W + k * NCH
        pltpu.sync_copy(src_hbm.at[pl.ds(base, NCH)], s_b)
        pltpu.sync_copy(dst_hbm.at[pl.ds(base, NCH)], d_b)
        pltpu.sync_copy(w_hbm.at[pl.ds(base, NCH)], w_b)

        def grp(g, c2):
            sl = pl.ds(g * 16, 16)
            a = plsc.load_gather(dis_v, [s_b[sl]])
            b = plsc.load_gather(dis_v, [d_b[sl]])
            n_b[sl] = -(a * w_b[sl] * b)
            return c2

        lax.fori_loop(0, NCH // 16, grp, 0)
        pltpu.sync_copy(n_b, out_hbm.at[pl.ds(base, NCH)])
        return carry

    lax.fori_loop(0, E_W // NCH, chunk, 0)


def _norm_sc(src3, ew3, src, dst, ew, z1d):
    return pl.kernel(
        _norm_body,
        out_type=jax.ShapeDtypeStruct((E,), jnp.float32),
        scratch_types=[
            pltpu.VMEM((NCHUNK, CH), jnp.int32),
            pltpu.VMEM((NCHUNK, CH), jnp.float32),
            pltpu.VMEM((NPAD,), jnp.float32),
            pltpu.VMEM((NCH,), jnp.int32),
            pltpu.VMEM((NCH,), jnp.int32),
            pltpu.VMEM((NCH,), jnp.float32),
            pltpu.VMEM((NCH,), jnp.float32),
            pltpu.VMEM_SHARED((NPAD,), jnp.float32),
            pltpu.SemaphoreType.DMA,
        ],
        **_PARAMS,
    )(src3, ew3, src, dst, ew, z1d)


_DNUMS = lax.GatherDimensionNumbers(
    offset_dims=(), collapsed_slice_dims=(0,), start_index_map=(0,))


def _bcast_lane(vec, lane):
    iv = jnp.full((16,), lane, jnp.int32)
    return lax.gather(vec, iv[:, None], _DNUMS, (1,),
                      mode=lax.GatherScatterMode.PROMISE_IN_BOUNDS)


def _spmm_body(y_hbm, src_hbm, dst_hbm, nrm_hbm, z2_hbm, out_hbm,
               sidx, didx, nrm, rows, acc, gsem, ssem, isem):
    cid = lax.axis_index("c")
    sid = lax.axis_index("s")
    wid = cid * NS + sid
    pltpu.sync_copy(z2_hbm.at[pl.ds(sid * ROWS_W, ROWS_W)],
                    acc.at[pl.ds(sid * ROWS_W, ROWS_W)])
    pltpu.sync_copy(src_hbm.at[wid, pl.ds(0, SB)], sidx.at[0])
    pltpu.sync_copy(dst_hbm.at[wid, pl.ds(0, SB)], didx.at[0])
    pltpu.sync_copy(nrm_hbm.at[wid, pl.ds(0, SB)], nrm.at[0])
    pltpu.async_copy(src_hbm.at[wid, pl.ds(SB, SB)], sidx.at[1], isem)
    pltpu.async_copy(dst_hbm.at[wid, pl.ds(SB, SB)], didx.at[1], isem)
    pltpu.async_copy(nrm_hbm.at[wid, pl.ds(SB, SB)], nrm.at[1], isem)
    plsc.subcore_barrier()

    def stage_wait():
        pltpu.make_async_copy(src_hbm.at[0, pl.ds(0, SB)], sidx.at[0],
                              isem).wait()
        pltpu.make_async_copy(dst_hbm.at[0, pl.ds(0, SB)], didx.at[0],
                              isem).wait()
        pltpu.make_async_copy(nrm_hbm.at[0, pl.ds(0, SB)], nrm.at[0],
                              isem).wait()

    def gather(pp, b16, b):
        pltpu.async_copy(y_hbm.at[sidx.at[pp, b16]], rows.at[b], gsem)

    def gather_wait(b):
        pltpu.make_async_copy(y_hbm.at[sidx.at[0, 0]], rows.at[b],
                              gsem).wait()

    def scatter(pp, b16, b):
        pltpu.async_copy(rows.at[b], acc.at[didx.at[pp, b16]], ssem,
                         add=True)

    def scatter_wait(b):
        pltpu.make_async_copy(rows.at[b], acc.at[didx.at[0, 0]], ssem).wait()

    gather(0, 0, 0)
    gather(0, 1, 1)

    def outer(oo, carry):
        for pp in range(2):
            q = 2 * oo + pp
            for b16 in range(SB):
                k = q * SB + b16
                b = b16 % NBUF
                gather_wait(b)

                def grp(g, c2, pp=pp, b16=b16, b=b):
                    nvec = nrm[pp, b16, pl.ds(g * 16, 16)]
                    base = g * 16
                    for l in range(16):
                        nv = nvec[l]
                        e = base + l
                        for j in range(F // 16):
                            sl = pl.ds(j * 16, 16)
                            rows[b, e, sl] = rows[b, e, sl] * nv
                    return c2

                lax.fori_loop(0, CH // 16, grp, 0)

                if b16 == 2:
                    @pl.when(jnp.logical_and(q >= 1, q <= NQ - 2))
                    def _(q=q, pp=pp):
                        blk = pl.ds((q + 1) * SB, SB)
                        pltpu.async_copy(src_hbm.at[wid, blk],
                                         sidx.at[1 - pp], isem)
                        pltpu.async_copy(dst_hbm.at[wid, blk],
                                         didx.at[1 - pp], isem)
                        pltpu.async_copy(nrm_hbm.at[wid, blk],
                                         nrm.at[1 - pp], isem)

                if b16 == SB - 3:
                    @pl.when(q <= NQ - 2)
                    def _():
                        stage_wait()

                @pl.when(k >= 2)
                def _(b16=b16):
                    scatter_wait((b16 + 2) % NBUF)

                @pl.when(k <= NCHUNK - 3)
                def _(pp=pp, b16=b16):
                    pp2 = pp if b16 < SB - 2 else 1 - pp
                    gather(pp2, (b16 + 2) % SB, (b16 + 2) % NBUF)

                scatter(pp, b16, b)
        return carry

    lax.fori_loop(0, NQ // 2, outer, 0)
    scatter_wait((NCHUNK - 2) % NBUF)
    scatter_wait((NCHUNK - 1) % NBUF)
    plsc.subcore_barrier()
    pltpu.sync_copy(acc.at[pl.ds(sid * ROWS_W, ROWS_W)],
                    out_hbm.at[cid, pl.ds(sid * ROWS_W, ROWS_W)])


def _spmm_sc(y, src3, dst3, nrm3, z2d):
    return pl.kernel(
        _spmm_body,
        out_type=jax.ShapeDtypeStruct((NC, NPAD, F), jnp.float32),
        scratch_types=[
            pltpu.VMEM((2, SB, CH), jnp.int32),
            pltpu.VMEM((2, SB, CH), jnp.int32),
            pltpu.VMEM((2, SB, CH), jnp.float32),
            pltpu.VMEM((NBUF, CH, F), jnp.float32),
            pltpu.VMEM_SHARED((NPAD, F), jnp.float32),
            pltpu.SemaphoreType.DMA,
            pltpu.SemaphoreType.DMA,
            pltpu.SemaphoreType.DMA,
        ],
        **_PARAMS,
    )(y, src3, dst3, nrm3, z2d)


BR = 2048


def _comb1_body(a_ref, out_ref):
    out_ref[...] = a_ref[0] + a_ref[1]


def _comb1_tc(a):
    return pl.pallas_call(
        _comb1_body,
        grid=(NPAD // BR,),
        in_specs=[pl.BlockSpec((NC, BR, F), lambda i: (0, i, 0))],
        out_specs=pl.BlockSpec((BR, F), lambda i: (i, 0)),
        out_shape=jax.ShapeDtypeStruct((NPAD, F), jnp.float32),
    )(a)


def _comb2_body(a_ref, x_ref, out_ref):
    out_ref[...] = 2.0 * (a_ref[0] + a_ref[1]) - x_ref[...]


def _comb2_tc(a, x):
    return pl.pallas_call(
        _comb2_body,
        grid=(NPAD // BR,),
        in_specs=[pl.BlockSpec((NC, BR, F), lambda i: (0, i, 0)),
                  pl.BlockSpec((BR, F), lambda i: (i, 0))],
        out_specs=pl.BlockSpec((BR, F), lambda i: (i, 0)),
        out_shape=jax.ShapeDtypeStruct((NPAD, F), jnp.float32),
    )(a, x)


def _final_body(x_ref, t1_ref, t2_ref, a3_ref, wz_ref, wh_ref, wl_ref,
                bz_ref, bh_ref, bl_ref, lin_ref, lg_ref):
    t1 = t1_ref[...]
    t3 = 2.0 * (a3_ref[0] + a3_ref[1]) - t1
    xcat = jnp.concatenate([x_ref[...], t1, t2_ref[...], t3], axis=1)
    zpre = jnp.dot(xcat, wz_ref[...], preferred_element_type=jnp.float32)
    hpre = jnp.dot(xcat, wh_ref[...], preferred_element_type=jnp.float32)
    z = jax.nn.sigmoid(zpre + bz_ref[...])
    ht = jnp.tanh(hpre + bh_ref[...])
    o = jax.nn.relu((1.0 - z) * ht)
    lin_ref[...] = (jnp.dot(o, wl_ref[...], preferred_element_type=jnp.float32)
                    + bl_ref[...])
    lg_ref[...] = jnp.broadcast_to(
        jnp.sqrt(jnp.sum(o, axis=1, keepdims=True)), o.shape)


def _final_tc(x, t1, t2, a3, wz, wh, wlp, bz, bh, blp):
    row = lambda i: (i, 0)
    full = lambda i: (0, 0)
    return pl.pallas_call(
        _final_body,
        grid=(NPAD // BR,),
        in_specs=[pl.BlockSpec((BR, F), row),
                  pl.BlockSpec((BR, F), row),
                  pl.BlockSpec((BR, F), row),
                  pl.BlockSpec((NC, BR, F), lambda i: (0, i, 0)),
                  pl.BlockSpec((4 * F, F), full),
                  pl.BlockSpec((4 * F, F), full),
                  pl.BlockSpec((F, F), full),
                  pl.BlockSpec((1, F), full),
                  pl.BlockSpec((1, F), full),
                  pl.BlockSpec((1, F), full)],
        out_specs=[pl.BlockSpec((BR, F), row), pl.BlockSpec((BR, F), row)],
        out_shape=[jax.ShapeDtypeStruct((NPAD, F), jnp.float32),
                   jax.ShapeDtypeStruct((NPAD, F), jnp.float32)],
    )(x, t1, t2, a3, wz, wh, wlp, bz, bh, blp)


def kernel(x, edge_index, edge_weight, W_xz, b_xz, W_hz, b_hz, W_xr, b_xr,
           W_hr, b_hr, W_xh, b_xh, W_hh, b_hh, W_lin, b_lin):
    src = edge_index[0]
    dst = edge_index[1]
    pad_idx = (jnp.arange(NW * EPAD, dtype=jnp.int32) * 37 % N
               ).reshape(NW, EPAD)
    zpad = jnp.zeros((NW, EPAD), jnp.float32)
    pad3 = lambda a, p: jnp.concatenate(
        [a.reshape(NW, E_W), p], axis=1).reshape(NW, NCHUNK, CH)
    src3 = pad3(src, pad_idx)
    dst3 = pad3(dst, pad_idx)
    ew3 = pad3(edge_weight, zpad)
    z1d = jnp.zeros((NPAD,), jnp.float32)
    z2d = jnp.zeros((NPAD, F), jnp.float32)
    xp = jnp.zeros((NPAD, F), jnp.float32).at[:N].set(x)

    nrm = _norm_sc(src3, ew3, src, dst, edge_weight, z1d)
    nrm3 = pad3(nrm, zpad)

    a1 = _spmm_sc(xp, src3, dst3, nrm3, z2d)
    t1 = _comb1_tc(a1)
    a2 = _spmm_sc(t1, src3, dst3, nrm3, z2d)
    t2 = _comb2_tc(a2, xp)
    a3 = _spmm_sc(t2, src3, dst3, nrm3, z2d)

    wz = W_xz.reshape(4 * F, F)
    wh = W_xh.reshape(4 * F, F)
    wlp = jnp.zeros((F, F), jnp.float32).at[:, :1].set(W_lin)
    bz = (b_xz + b_hz).reshape(1, F)
    bh = (b_xh + b_hh).reshape(1, F)
    blp = jnp.broadcast_to(b_lin, (1, F))

    lin_full, lg_full = _final_tc(xp, t1, t2, a3, wz, wh, wlp, bz, bh, blp)
    return (lin_full[:N, :1], lg_full[:N, 0])


# speedup vs baseline: 1.1025x; 1.1025x over previous
import jax
import jax.numpy as jnp
from jax import lax
from jax.experimental import pallas as pl
from jax.experimental.pallas import tpu as pltpu
from jax.experimental.pallas import tpu_sc as plsc

N = 10000
E = 320000
F = 128
NPAD = 10240
NC = 2
NS = 16
NW = NC * NS
E_W = E // NW
CH = 80
EPAD = 240
NCHUNK = (E_W + EPAD) // CH
SB = 8
NQ = NCHUNK // SB
NBUF = 4
ROWS_W = NPAD // NS
DEG_W = NPAD // NS

_MESH = dict(core_axis_name="c", subcore_axis_name="s", num_cores=NC,
             num_subcores=NS)
_PARAMS = dict(
    mesh=plsc.VectorSubcoreMesh(**_MESH),
    compiler_params=pltpu.CompilerParams(needs_layout_passes=False),
)


def _deg_body(src_hbm, w_hbm, z1_hbm, out_hbm, idx_b, w_b, degsh, ssem):
    cid = lax.axis_index("c")
    sid = lax.axis_index("s")
    wid = cid * NS + sid
    pltpu.sync_copy(z1_hbm.at[pl.ds(sid * DEG_W, DEG_W)],
                    degsh.at[pl.ds(sid * DEG_W, DEG_W)])
    pltpu.sync_copy(src_hbm.at[wid], idx_b)
    pltpu.sync_copy(w_hbm.at[wid], w_b)
    plsc.subcore_barrier()

    def fire(k, carry):
        pltpu.async_copy(w_b.at[k], degsh.at[idx_b.at[k]], ssem, add=True)
        return carry

    lax.fori_loop(0, NCHUNK, fire, 0)

    def drain(k, carry):
        pltpu.make_async_copy(w_b.at[0], degsh.at[idx_b.at[0]], ssem).wait()
        return carry

    lax.fori_loop(0, NCHUNK, drain, 0)
    plsc.subcore_barrier()
    pltpu.sync_copy(degsh.at[pl.ds(sid * DEG_W, DEG_W)],
                    out_hbm.at[cid, pl.ds(sid * DEG_W, DEG_W)])


def _deg_partials(src3, ew3, z1d):
    return pl.kernel(
        _deg_body,
        out_type=jax.ShapeDtypeStruct((NC, NPAD), jnp.float32),
        scratch_types=[
            pltpu.VMEM((NCHUNK, CH), jnp.int32),
            pltpu.VMEM((NCHUNK, CH), jnp.float32),
            pltpu.VMEM_SHARED((NPAD,), jnp.float32),
            pltpu.SemaphoreType.DMA,
        ],
        **_PARAMS,
    )(src3, ew3, z1d)


def _dis_body(degp_ref, dis_ref):
    deg = degp_ref[0:1, :] + degp_ref[1:2, :]
    safe = jnp.where(deg > 0, deg, 1.0)
    dis_ref[...] = jnp.where(deg > 0, lax.rsqrt(safe), 0.0)


def _dis_tc(deg_partials):
    out = pl.pallas_call(
        _dis_body,
        out_shape=jax.ShapeDtypeStruct((1, NPAD), jnp.float32),
    )(deg_partials)
    return out.reshape(NPAD)


NCH = 2000


def _norm_body(dis_hbm, src_hbm, dst_hbm, w_hbm, out_hbm,
               dis_v, s_b, d_b, w_b, n_b):
    cid = lax.axis_index("c")
    sid = lax.axis_index("s")
    wid = cid * NS + sid
    pltpu.sync_copy(dis_hbm, dis_v)

    def chunk(k, carry):
        base = wid * E_W + k * NCH
        pltpu.sync_copy(src_hbm.at[pl.ds(base, NCH)], s_b)
        pltpu.sync_copy(dst_hbm.at[pl.ds(base, NCH)], d_b)
        pltpu.sync_copy(w_hbm.at[pl.ds(base, NCH)], w_b)

        def grp(g, c2):
            sl = pl.ds(g * 16, 16)
            a = plsc.load_gather(dis_v, [s_b[sl]])
            b = plsc.load_gather(dis_v, [d_b[sl]])
            n_b[sl] = -(a * w_b[sl] * b)
            return c2

        lax.fori_loop(0, NCH // 16, grp, 0)
        pltpu.sync_copy(n_b, out_hbm.at[pl.ds(base, NCH)])
        return carry

    lax.fori_loop(0, E_W // NCH, chunk, 0)


def _norm_sc(dis, src, dst, ew):
    return pl.kernel(
        _norm_body,
        out_type=jax.ShapeDtypeStruct((E,), jnp.float32),
        scratch_types=[
            pltpu.VMEM((NPAD,), jnp.float32),
            pltpu.VMEM((NCH,), jnp.int32),
            pltpu.VMEM((NCH,), jnp.int32),
            pltpu.VMEM((NCH,), jnp.float32),
            pltpu.VMEM((NCH,), jnp.float32),
        ],
        **_PARAMS,
    )(dis, src, dst, ew)


_DNUMS = lax.GatherDimensionNumbers(
    offset_dims=(), collapsed_slice_dims=(0,), start_index_map=(0,))


def _bcast_lane(vec, lane):
    iv = jnp.full((16,), lane, jnp.int32)
    return lax.gather(vec, iv[:, None], _DNUMS, (1,),
                      mode=lax.GatherScatterMode.PROMISE_IN_BOUNDS)


def _spmm_body(y_hbm, src_hbm, dst_hbm, nrm_hbm, z2_hbm, out_hbm,
               sidx, didx, nrm, rows, acc, gsem, ssem, isem):
    cid = lax.axis_index("c")
    sid = lax.axis_index("s")
    wid = cid * NS + sid
    pltpu.sync_copy(z2_hbm.at[pl.ds(sid * ROWS_W, ROWS_W)],
                    acc.at[pl.ds(sid * ROWS_W, ROWS_W)])
    pltpu.sync_copy(src_hbm.at[wid, pl.ds(0, SB)], sidx.at[0])
    pltpu.sync_copy(dst_hbm.at[wid, pl.ds(0, SB)], didx.at[0])
    pltpu.sync_copy(nrm_hbm.at[wid, pl.ds(0, SB)], nrm.at[0])
    pltpu.async_copy(src_hbm.at[wid, pl.ds(SB, SB)], sidx.at[1], isem)
    pltpu.async_copy(dst_hbm.at[wid, pl.ds(SB, SB)], didx.at[1], isem)
    pltpu.async_copy(nrm_hbm.at[wid, pl.ds(SB, SB)], nrm.at[1], isem)
    plsc.subcore_barrier()

    def stage_wait():
        pltpu.make_async_copy(src_hbm.at[0, pl.ds(0, SB)], sidx.at[0],
                              isem).wait()
        pltpu.make_async_copy(dst_hbm.at[0, pl.ds(0, SB)], didx.at[0],
                              isem).wait()
        pltpu.make_async_copy(nrm_hbm.at[0, pl.ds(0, SB)], nrm.at[0],
                              isem).wait()

    def gather(pp, b16, b):
        pltpu.async_copy(y_hbm.at[sidx.at[pp, b16]], rows.at[b], gsem)

    def gather_wait(b):
        pltpu.make_async_copy(y_hbm.at[sidx.at[0, 0]], rows.at[b],
                              gsem).wait()

    def scatter(pp, b16, b):
        pltpu.async_copy(rows.at[b], acc.at[didx.at[pp, b16]], ssem,
                         add=True)

    def scatter_wait(b):
        pltpu.make_async_copy(rows.at[b], acc.at[didx.at[0, 0]], ssem).wait()

    gather(0, 0, 0)
    gather(0, 1, 1)

    def outer(oo, carry):
        for pp in range(2):
            q = 2 * oo + pp
            for b16 in range(SB):
                k = q * SB + b16
                b = b16 % NBUF
                gather_wait(b)

                def grp(g, c2, pp=pp, b16=b16, b=b):
                    nvec = nrm[pp, b16, pl.ds(g * 16, 16)]
                    base = g * 16
                    for l in range(16):
                        nv = nvec[l]
                        e = base + l
                        for j in range(F // 16):
                            sl = pl.ds(j * 16, 16)
                            rows[b, e, sl] = rows[b, e, sl] * nv
                    return c2

                lax.fori_loop(0, CH // 16, grp, 0)

                if b16 == 2:
                    @pl.when(jnp.logical_and(q >= 1, q <= NQ - 2))
                    def _(q=q, pp=pp):
                        blk = pl.ds((q + 1) * SB, SB)
                        pltpu.async_copy(src_hbm.at[wid, blk],
                                         sidx.at[1 - pp], isem)
                        pltpu.async_copy(dst_hbm.at[wid, blk],
                                         didx.at[1 - pp], isem)
                        pltpu.async_copy(nrm_hbm.at[wid, blk],
                                         nrm.at[1 - pp], isem)

                if b16 == SB - 3:
                    @pl.when(q <= NQ - 2)
                    def _():
                        stage_wait()

                @pl.when(k >= 2)
                def _(b16=b16):
                    scatter_wait((b16 + 2) % NBUF)

                @pl.when(k <= NCHUNK - 3)
                def _(pp=pp, b16=b16):
                    pp2 = pp if b16 < SB - 2 else 1 - pp
                    gather(pp2, (b16 + 2) % SB, (b16 + 2) % NBUF)

                scatter(pp, b16, b)
        return carry

    lax.fori_loop(0, NQ // 2, outer, 0)
    scatter_wait((NCHUNK - 2) % NBUF)
    scatter_wait((NCHUNK - 1) % NBUF)
    plsc.subcore_barrier()
    pltpu.sync_copy(acc.at[pl.ds(sid * ROWS_W, ROWS_W)],
                    out_hbm.at[cid, pl.ds(sid * ROWS_W, ROWS_W)])


def _spmm_sc(y, src3, dst3, nrm3, z2d):
    return pl.kernel(
        _spmm_body,
        out_type=jax.ShapeDtypeStruct((NC, NPAD, F), jnp.float32),
        scratch_types=[
            pltpu.VMEM((2, SB, CH), jnp.int32),
            pltpu.VMEM((2, SB, CH), jnp.int32),
            pltpu.VMEM((2, SB, CH), jnp.float32),
            pltpu.VMEM((NBUF, CH, F), jnp.float32),
            pltpu.VMEM_SHARED((NPAD, F), jnp.float32),
            pltpu.SemaphoreType.DMA,
            pltpu.SemaphoreType.DMA,
            pltpu.SemaphoreType.DMA,
        ],
        **_PARAMS,
    )(y, src3, dst3, nrm3, z2d)


BR = 2048


def _comb1_body(a_ref, out_ref):
    out_ref[...] = a_ref[0] + a_ref[1]


def _comb1_tc(a):
    return pl.pallas_call(
        _comb1_body,
        grid=(NPAD // BR,),
        in_specs=[pl.BlockSpec((NC, BR, F), lambda i: (0, i, 0))],
        out_specs=pl.BlockSpec((BR, F), lambda i: (i, 0)),
        out_shape=jax.ShapeDtypeStruct((NPAD, F), jnp.float32),
    )(a)


def _comb2_body(a_ref, x_ref, out_ref):
    out_ref[...] = 2.0 * (a_ref[0] + a_ref[1]) - x_ref[...]


def _comb2_tc(a, x):
    return pl.pallas_call(
        _comb2_body,
        grid=(NPAD // BR,),
        in_specs=[pl.BlockSpec((NC, BR, F), lambda i: (0, i, 0)),
                  pl.BlockSpec((BR, F), lambda i: (i, 0))],
        out_specs=pl.BlockSpec((BR, F), lambda i: (i, 0)),
        out_shape=jax.ShapeDtypeStruct((NPAD, F), jnp.float32),
    )(a, x)


def _final_body(x_ref, t1_ref, t2_ref, a3_ref, wz_ref, wh_ref, wl_ref,
                bz_ref, bh_ref, bl_ref, lin_ref, lg_ref):
    t1 = t1_ref[...]
    t3 = 2.0 * (a3_ref[0] + a3_ref[1]) - t1
    xcat = jnp.concatenate([x_ref[...], t1, t2_ref[...], t3], axis=1)
    zpre = jnp.dot(xcat, wz_ref[...], preferred_element_type=jnp.float32)
    hpre = jnp.dot(xcat, wh_ref[...], preferred_element_type=jnp.float32)
    z = jax.nn.sigmoid(zpre + bz_ref[...])
    ht = jnp.tanh(hpre + bh_ref[...])
    o = jax.nn.relu((1.0 - z) * ht)
    lin_ref[...] = (jnp.dot(o, wl_ref[...], preferred_element_type=jnp.float32)
                    + bl_ref[...])
    lg_ref[...] = jnp.broadcast_to(
        jnp.sqrt(jnp.sum(o, axis=1, keepdims=True)), o.shape)


def _final_tc(x, t1, t2, a3, wz, wh, wlp, bz, bh, blp):
    row = lambda i: (i, 0)
    full = lambda i: (0, 0)
    return pl.pallas_call(
        _final_body,
        grid=(NPAD // BR,),
        in_specs=[pl.BlockSpec((BR, F), row),
                  pl.BlockSpec((BR, F), row),
                  pl.BlockSpec((BR, F), row),
                  pl.BlockSpec((NC, BR, F), lambda i: (0, i, 0)),
                  pl.BlockSpec((4 * F, F), full),
                  pl.BlockSpec((4 * F, F), full),
                  pl.BlockSpec((F, F), full),
                  pl.BlockSpec((1, F), full),
                  pl.BlockSpec((1, F), full),
                  pl.BlockSpec((1, F), full)],
        out_specs=[pl.BlockSpec((BR, F), row), pl.BlockSpec((BR, F), row)],
        out_shape=[jax.ShapeDtypeStruct((NPAD, F), jnp.float32),
                   jax.ShapeDtypeStruct((NPAD, F), jnp.float32)],
    )(x, t1, t2, a3, wz, wh, wlp, bz, bh, blp)


def kernel(x, edge_index, edge_weight, W_xz, b_xz, W_hz, b_hz, W_xr, b_xr,
           W_hr, b_hr, W_xh, b_xh, W_hh, b_hh, W_lin, b_lin):
    src = edge_index[0]
    dst = edge_index[1]
    pad_idx = (jnp.arange(NW * EPAD, dtype=jnp.int32) * 37 % N
               ).reshape(NW, EPAD)
    zpad = jnp.zeros((NW, EPAD), jnp.float32)
    pad3 = lambda a, p: jnp.concatenate(
        [a.reshape(NW, E_W), p], axis=1).reshape(NW, NCHUNK, CH)
    src3 = pad3(src, pad_idx)
    dst3 = pad3(dst, pad_idx)
    ew3 = pad3(edge_weight, zpad)
    z1d = jnp.zeros((NPAD,), jnp.float32)
    z2d = jnp.zeros((NPAD, F), jnp.float32)
    xp = jnp.zeros((NPAD, F), jnp.float32).at[:N].set(x)

    degp = _deg_partials(src3, ew3, z1d)
    dis = _dis_tc(degp)
    nrm = _norm_sc(dis, src, dst, edge_weight)
    nrm3 = pad3(nrm, zpad)

    a1 = _spmm_sc(xp, src3, dst3, nrm3, z2d)
    t1 = _comb1_tc(a1)
    a2 = _spmm_sc(t1, src3, dst3, nrm3, z2d)
    t2 = _comb2_tc(a2, xp)
    a3 = _spmm_sc(t2, src3, dst3, nrm3, z2d)

    wz = W_xz.reshape(4 * F, F)
    wh = W_xh.reshape(4 * F, F)
    wlp = jnp.zeros((F, F), jnp.float32).at[:, :1].set(W_lin)
    bz = (b_xz + b_hz).reshape(1, F)
    bh = (b_xh + b_hh).reshape(1, F)
    blp = jnp.broadcast_to(b_lin, (1, F))

    lin_full, lg_full = _final_tc(xp, t1, t2, a3, wz, wh, wlp, bz, bh, blp)
    return (lin_full[:N, :1], lg_full[:N, 0])
